# Initial kernel scaffold; baseline (speedup 1.0000x reference)
import jax
import jax.numpy as jnp
from jax.experimental import pallas as pl


def kernel(initial_predictions, signals):
    raise NotImplementedError("write your pallas kernel here")


# speedup vs baseline: 3.3157x; 3.3157x over previous
import functools

import jax
import jax.numpy as jnp
from jax import lax
from jax.experimental import pallas as pl
from jax.experimental.pallas import tpu as pltpu
from jax.experimental.pallas import tpu_sc as plsc

SIGNAL_LENGTH = 8192
BASE_STEP_SIZE = 0.002
MAX_ITERATIONS = 7
EPS = 0.01

NC = 2
NS = 16
L = 16
NW = NC * NS

IDX_CHUNK = 128


def _interp(wr, vl, vr):
    return (1.0 - wr) * vl + wr * vr


def _refine_body(n_chains, pos_hbm, sig_hbm, out_hbm,
                 pos_v, base_v, wr_v, idx_v, val_v, sem):
    ch_per_w = n_chains // NW
    vregs = ch_per_w // L
    n_idx = 6 * ch_per_w
    n_chunks = n_idx // IDX_CHUNK

    wid = lax.axis_index("s") * NC + lax.axis_index("c")
    first = wid * ch_per_w

    pltpu.sync_copy(pos_hbm.at[pl.ds(first * 1, ch_per_w)], pos_v)

    lane = lax.iota(jnp.int32, L)
    for j in range(vregs):
        n = first + j * L + lane
        base_v[pl.ds(j * L, L)] = lax.div(n, 3) * SIGNAL_LENGTH

    def one_iteration(t, carry):
        for j in range(vregs):
            p = pos_v[pl.ds(j * L, L)]
            b = base_v[pl.ds(j * L, L)]
            for si, off in enumerate((-EPS, 0.0, EPS)):
                ps = p * 8191.0 if si == 1 else (p + off) * 8191.0
                fi = ps.astype(jnp.int32)
                ff = fi.astype(jnp.float32)
                il = jnp.minimum(jnp.maximum(fi, 0), SIGNAL_LENGTH - 1)
                ci = jnp.where(ps > ff, fi + 1, fi)
                ir = jnp.minimum(jnp.maximum(ci, 0), SIGNAL_LENGTH - 1)
                wr = ps - il.astype(jnp.float32)
                fl_l = (si * 2) * ch_per_w + j * L
                fl_r = (si * 2 + 1) * ch_per_w + j * L
                idx_v[fl_l // IDX_CHUNK, pl.ds(fl_l % IDX_CHUNK, L)] = il + b
                idx_v[fl_r // IDX_CHUNK, pl.ds(fl_r % IDX_CHUNK, L)] = ir + b
                wr_v[pl.ds(si * ch_per_w + j * L, L)] = wr

        copies = [
            pltpu.async_copy(sig_hbm.at[idx_v.at[ch]], val_v.at[ch], sem)
            for ch in range(n_chunks)
        ]
        for cp in copies:
            cp.wait()

        for j in range(vregs):
            v6 = []
            for g in range(6):
                fl = g * ch_per_w + j * L
                v6.append(val_v[fl // IDX_CHUNK, pl.ds(fl % IDX_CHUNK, L)])
            w3 = [wr_v[pl.ds(si * ch_per_w + j * L, L)] for si in range(3)]
            v_minus = _interp(w3[0], v6[0], v6[1])
            v_mid = _interp(w3[1], v6[2], v6[3])
            v_plus = _interp(w3[2], v6[4], v6[5])
            grad = (v_plus - v_minus) / (2 * EPS)
            curv = (v_plus + v_minus - 2 * v_mid) / (EPS * EPS)
            step = -grad / (curv + 1e-6)
            p = pos_v[pl.ds(j * L, L)]
            p = p + BASE_STEP_SIZE * step
            p = jnp.minimum(jnp.maximum(p, 0.0), 1.0)
            pos_v[pl.ds(j * L, L)] = p
        return carry

    lax.fori_loop(0, MAX_ITERATIONS, one_iteration, 0)

    pltpu.sync_copy(pos_v, out_hbm.at[pl.ds(first * 1, ch_per_w)])


def kernel(initial_predictions, signals):
    if signals.ndim == 3:
        signals = jnp.squeeze(signals, axis=1)
    batch, num_peaks = initial_predictions.shape
    n_chains = batch * num_peaks
    ch_per_w = n_chains // NW

    pos_flat = initial_predictions.reshape(-1)
    sig_flat = signals.reshape(-1)

    mesh = plsc.VectorSubcoreMesh(core_axis_name="c", subcore_axis_name="s")
    run = pl.kernel(
        functools.partial(_refine_body, n_chains),
        out_type=jax.ShapeDtypeStruct((n_chains,), jnp.float32),
        mesh=mesh,
        scratch_types=[
            pltpu.VMEM((ch_per_w,), jnp.float32),
            pltpu.VMEM((ch_per_w,), jnp.int32),
            pltpu.VMEM((3 * ch_per_w,), jnp.float32),
            pltpu.VMEM((6 * ch_per_w // IDX_CHUNK, IDX_CHUNK), jnp.int32),
            pltpu.VMEM((6 * ch_per_w // IDX_CHUNK, IDX_CHUNK), jnp.float32),
            pltpu.SemaphoreType.DMA,
        ],
    )
    out = run(pos_flat, sig_flat)
    return out.reshape(batch, num_peaks)


# speedup vs baseline: 5.2522x; 1.5840x over previous
import functools

import jax
import jax.numpy as jnp
from jax import lax
from jax.experimental import pallas as pl
from jax.experimental.pallas import tpu as pltpu
from jax.experimental.pallas import tpu_sc as plsc

SIGNAL_LENGTH = 8192
BASE_STEP_SIZE = 0.002
MAX_ITERATIONS = 7
EPS = 0.01

NC = 2
NS = 16
L = 16
NW = NC * NS

SLAB_ROWS = 8
CHAINS_PER_SLAB = 3 * SLAB_ROWS


def _refine_body(n_slabs_per_w, pos_hbm, sig_hbm, out_hbm, pos_v, buf, sem):
    n_rows, n_cols = sig_hbm.shape
    sig3 = sig_hbm.reshape(n_rows // SLAB_ROWS, SLAB_ROWS, n_cols)

    wid = lax.axis_index("s") * NC + lax.axis_index("c")
    lane = lax.iota(jnp.int32, L)

    rows = [lax.div(v * 8 + lane, 3) for v in range(2)]

    def sample_value(p, off, rs):
        ps = p * 8191.0 if off == 0.0 else (p + off) * 8191.0
        fi = ps.astype(jnp.int32)
        ff = fi.astype(jnp.float32)
        il = jnp.minimum(jnp.maximum(fi, 0), SIGNAL_LENGTH - 1)
        ci = jnp.where(ps > ff, fi + 1, fi)
        ir = jnp.minimum(jnp.maximum(ci, 0), SIGNAL_LENGTH - 1)
        wr = ps - il.astype(jnp.float32)
        vl = plsc.load_gather(buf, [rs, il])
        vr = plsc.load_gather(buf, [rs, ir])
        return (1.0 - wr) * vl + wr * vr

    def newton_iter(p, rs):
        v_mid = sample_value(p, 0.0, rs)
        v_minus = sample_value(p, -EPS, rs)
        v_plus = sample_value(p, EPS, rs)
        grad = (v_plus - v_minus) / (2 * EPS)
        curv = (v_plus + v_minus - 2 * v_mid) / (EPS * EPS)
        step = -grad / (curv + 1e-6)
        p = p + BASE_STEP_SIZE * step
        return jnp.minimum(jnp.maximum(p, 0.0), 1.0)

    def do_slab(g, carry):
        slab = wid * n_slabs_per_w + g
        pltpu.async_copy(sig3.at[slab], buf, sem).wait()
        pltpu.sync_copy(pos_hbm.at[pl.ds(slab * CHAINS_PER_SLAB,
                                         CHAINS_PER_SLAB)], pos_v)
        p0 = pos_v[pl.ds(0, L)]
        p1 = pos_v[pl.ds(8, L)]
        for _ in range(MAX_ITERATIONS):
            p0 = newton_iter(p0, rows[0])
            p1 = newton_iter(p1, rows[1])
        pos_v[pl.ds(0, L)] = p0
        pos_v[pl.ds(8, L)] = p1
        pltpu.sync_copy(pos_v, out_hbm.at[pl.ds(slab * CHAINS_PER_SLAB,
                                                CHAINS_PER_SLAB)])
        return carry

    lax.fori_loop(0, n_slabs_per_w, do_slab, 0)


def kernel(initial_predictions, signals):
    if signals.ndim == 3:
        signals = jnp.squeeze(signals, axis=1)
    batch, num_peaks = initial_predictions.shape
    n_chains = batch * num_peaks
    n_slabs_per_w = batch // (NW * SLAB_ROWS)

    pos_flat = initial_predictions.reshape(-1)

    mesh = plsc.VectorSubcoreMesh(core_axis_name="c", subcore_axis_name="s")
    run = pl.kernel(
        functools.partial(_refine_body, n_slabs_per_w),
        out_type=jax.ShapeDtypeStruct((n_chains,), jnp.float32),
        mesh=mesh,
        compiler_params=pltpu.CompilerParams(use_tc_tiling_on_sc=True,
                                             needs_layout_passes=False),
        scratch_types=[
            pltpu.VMEM((CHAINS_PER_SLAB,), jnp.float32),
            pltpu.VMEM((SLAB_ROWS, SIGNAL_LENGTH), jnp.float32),
            pltpu.SemaphoreType.DMA,
        ],
    )
    out = run(pos_flat, signals)
    return out.reshape(batch, num_peaks)


# speedup vs baseline: 6.2974x; 1.1990x over previous
import functools

import jax
import jax.numpy as jnp
from jax import lax
from jax.experimental import pallas as pl
from jax.experimental.pallas import tpu as pltpu
from jax.experimental.pallas import tpu_sc as plsc

SIGNAL_LENGTH = 8192
BASE_STEP_SIZE = 0.002
MAX_ITERATIONS = 7
EPS = 0.01

NC = 2
NS = 16
L = 16
NW = NC * NS

SLAB_ROWS = 8
CHAINS_PER_SLAB = 3 * SLAB_ROWS
HALF_ROWS = SLAB_ROWS // 2


def _refine_body(n_slabs_per_w, pos_hbm, sig_hbm, out_hbm,
                 pos_v, buf0, buf1, buf2, sem):
    n_rows, n_cols = sig_hbm.shape
    sig3 = sig_hbm.reshape(n_rows // SLAB_ROWS, SLAB_ROWS, n_cols)
    bufs = (buf0, buf1, buf2)

    wid = lax.axis_index("s") * NC + lax.axis_index("c")
    first_slab = wid * n_slabs_per_w
    lane = lax.iota(jnp.int32, L)
    rs = jnp.minimum(lax.div(lane, 3), HALF_ROWS - 1)
    live = lane < 12

    def start_half(g, parity, buf):
        return pltpu.async_copy(
            sig3.at[first_slab + g, pl.ds(parity * HALF_ROWS, HALF_ROWS)],
            buf, sem)

    def wait_half():
        pltpu.make_async_copy(sig3.at[0, pl.ds(0, HALF_ROWS)],
                              bufs[0], sem).wait()

    def sample_value(p, off, buf):
        ps = p * 8191.0 if off == 0.0 else (p + off) * 8191.0
        fi = ps.astype(jnp.int32)
        ff = fi.astype(jnp.float32)
        il = jnp.minimum(jnp.maximum(fi, 0), SIGNAL_LENGTH - 1)
        ci = jnp.where(ps > ff, fi + 1, fi)
        ir = jnp.minimum(jnp.maximum(ci, 0), SIGNAL_LENGTH - 1)
        wr = ps - il.astype(jnp.float32)
        vl = plsc.load_gather(buf, [rs, il])
        vr = plsc.load_gather(buf, [rs, ir])
        return (1.0 - wr) * vl + wr * vr

    def newton_chain(p, buf):
        for _ in range(MAX_ITERATIONS):
            v_mid = sample_value(p, 0.0, buf)
            v_minus = sample_value(p, -EPS, buf)
            v_plus = sample_value(p, EPS, buf)
            grad = (v_plus - v_minus) / (2 * EPS)
            curv = (v_plus + v_minus - 2 * v_mid) / (EPS * EPS)
            step = -grad / (curv + 1e-6)
            p = p + BASE_STEP_SIZE * step
            p = jnp.minimum(jnp.maximum(p, 0.0), 1.0)
        return p

    def do_slab(g, top, bot, pre0, pre1):
        if pre0 is not None:
            start_half(*pre0)
        wait_half()
        wait_half()
        p0 = pos_v[pl.ds(g * CHAINS_PER_SLAB, L)]
        p1 = pos_v[pl.ds(g * CHAINS_PER_SLAB + 12, L)]
        p0 = newton_chain(p0, top)
        p1 = newton_chain(p1, bot)
        plsc.store_compressed(pos_v.at[pl.ds(g * CHAINS_PER_SLAB, L)],
                              p0, mask=live)
        plsc.store_compressed(pos_v.at[pl.ds(g * CHAINS_PER_SLAB + 12, L)],
                              p1, mask=live)
        if pre1 is not None:
            start_half(*pre1)

    pltpu.sync_copy(pos_hbm.at[pl.ds(wid * (n_slabs_per_w * CHAINS_PER_SLAB),
                                     n_slabs_per_w * CHAINS_PER_SLAB)],
                    pos_v.at[pl.ds(0, n_slabs_per_w * CHAINS_PER_SLAB)])

    start_half(0, 0, bufs[0])
    start_half(0, 1, bufs[1])

    def triple(k, carry):
        g = k * 3
        do_slab(g + 0, bufs[0], bufs[1],
                (g + 1, 0, bufs[2]), (g + 1, 1, bufs[0]))
        do_slab(g + 1, bufs[2], bufs[0],
                (g + 2, 0, bufs[1]), (g + 2, 1, bufs[2]))
        do_slab(g + 2, bufs[1], bufs[2],
                (g + 3, 0, bufs[0]), (g + 3, 1, bufs[1]))
        return carry

    lax.fori_loop(0, (n_slabs_per_w - 1) // 3, triple, 0)
    do_slab(n_slabs_per_w - 1, bufs[0], bufs[1], None, None)

    pltpu.sync_copy(pos_v.at[pl.ds(0, n_slabs_per_w * CHAINS_PER_SLAB)],
                    out_hbm.at[pl.ds(wid * (n_slabs_per_w * CHAINS_PER_SLAB),
                                     n_slabs_per_w * CHAINS_PER_SLAB)])


def kernel(initial_predictions, signals):
    if signals.ndim == 3:
        signals = jnp.squeeze(signals, axis=1)
    batch, num_peaks = initial_predictions.shape
    n_chains = batch * num_peaks
    n_slabs_per_w = batch // (NW * SLAB_ROWS)

    pos_flat = initial_predictions.reshape(-1)

    mesh = plsc.VectorSubcoreMesh(core_axis_name="c", subcore_axis_name="s")
    run = pl.kernel(
        functools.partial(_refine_body, n_slabs_per_w),
        out_type=jax.ShapeDtypeStruct((n_chains,), jnp.float32),
        mesh=mesh,
        compiler_params=pltpu.CompilerParams(use_tc_tiling_on_sc=True,
                                             needs_layout_passes=False),
        scratch_types=[
            pltpu.VMEM((n_slabs_per_w * CHAINS_PER_SLAB + L,), jnp.float32),
            pltpu.VMEM((HALF_ROWS, SIGNAL_LENGTH), jnp.float32),
            pltpu.VMEM((HALF_ROWS, SIGNAL_LENGTH), jnp.float32),
            pltpu.VMEM((HALF_ROWS, SIGNAL_LENGTH), jnp.float32),
            pltpu.SemaphoreType.DMA,
        ],
    )
    out = run(pos_flat, signals)
    return out.reshape(batch, num_peaks)


# speedup vs baseline: 6.5338x; 1.0375x over previous
import functools

import jax
import jax.numpy as jnp
from jax import lax
from jax.experimental import pallas as pl
from jax.experimental.pallas import tpu as pltpu
from jax.experimental.pallas import tpu_sc as plsc

SIGNAL_LENGTH = 8192
BASE_STEP_SIZE = 0.002
MAX_ITERATIONS = 7
EPS = 0.01

NC = 2
NS = 16
L = 16
NW = NC * NS

SLAB_ROWS = 8
CHAINS_PER_SLAB = 3 * SLAB_ROWS
HALF_ROWS = SLAB_ROWS // 2


def _refine_body(n_slabs_per_w, pos_hbm, sig_hbm, out_hbm,
                 pos_v, buf0, buf1, buf2, sem):
    n_rows, n_cols = sig_hbm.shape
    sig3 = sig_hbm.reshape(n_rows // SLAB_ROWS, SLAB_ROWS, n_cols)
    bufs = (buf0, buf1, buf2)

    wid = lax.axis_index("s") * NC + lax.axis_index("c")
    first_slab = wid * n_slabs_per_w
    lane = lax.iota(jnp.int32, L)
    rs = jnp.minimum(lax.div(lane, 3), HALF_ROWS - 1)
    live = lane < 12

    def start_half(g, parity, buf):
        return pltpu.async_copy(
            sig3.at[first_slab + g, pl.ds(parity * HALF_ROWS, HALF_ROWS)],
            buf, sem)

    def wait_half():
        pltpu.make_async_copy(sig3.at[0, pl.ds(0, HALF_ROWS)],
                              bufs[0], sem).wait()

    def sample_value(p, off, buf):
        ps = p * 8191.0 if off == 0.0 else (p + off) * 8191.0
        fi = ps.astype(jnp.int32)
        ff = fi.astype(jnp.float32)
        il = jnp.minimum(jnp.maximum(fi, 0), SIGNAL_LENGTH - 1)
        ci = jnp.where(ps > ff, fi + 1, fi)
        ir = jnp.minimum(jnp.maximum(ci, 0), SIGNAL_LENGTH - 1)
        wr = ps - il.astype(jnp.float32)
        vl = plsc.load_gather(buf, [rs, il])
        vr = plsc.load_gather(buf, [rs, ir])
        return (1.0 - wr) * vl + wr * vr

    def newton_chain(p, buf):
        for _ in range(MAX_ITERATIONS):
            v_mid = sample_value(p, 0.0, buf)
            v_minus = sample_value(p, -EPS, buf)
            v_plus = sample_value(p, EPS, buf)
            grad = (v_plus - v_minus) / (2 * EPS)
            curv = (v_plus + v_minus - 2 * v_mid) / (EPS * EPS)
            step = -grad / (curv + 1e-6)
            p = p + BASE_STEP_SIZE * step
            p = jnp.minimum(jnp.maximum(p, 0.0), 1.0)
        return p

    def do_slab(g, top, bot, pre0, pre1):
        if pre0 is not None:
            start_half(*pre0)
        wait_half()
        wait_half()
        p0 = pos_v[pl.ds(g * CHAINS_PER_SLAB, L)]
        p1 = pos_v[pl.ds(g * CHAINS_PER_SLAB + 12, L)]
        p0 = newton_chain(p0, top)
        if pre1 is not None:
            start_half(*pre1)
        p1 = newton_chain(p1, bot)
        plsc.store_compressed(pos_v.at[pl.ds(g * CHAINS_PER_SLAB, L)],
                              p0, mask=live)
        plsc.store_compressed(pos_v.at[pl.ds(g * CHAINS_PER_SLAB + 12, L)],
                              p1, mask=live)

    pltpu.sync_copy(pos_hbm.at[pl.ds(wid * (n_slabs_per_w * CHAINS_PER_SLAB),
                                     n_slabs_per_w * CHAINS_PER_SLAB)],
                    pos_v.at[pl.ds(0, n_slabs_per_w * CHAINS_PER_SLAB)])

    start_half(0, 0, bufs[0])
    start_half(0, 1, bufs[1])

    def triple(k, carry):
        g = k * 3
        do_slab(g + 0, bufs[0], bufs[1],
                (g + 1, 0, bufs[2]), (g + 1, 1, bufs[0]))
        do_slab(g + 1, bufs[2], bufs[0],
                (g + 2, 0, bufs[1]), (g + 2, 1, bufs[2]))
        do_slab(g + 2, bufs[1], bufs[2],
                (g + 3, 0, bufs[0]), (g + 3, 1, bufs[1]))
        return carry

    lax.fori_loop(0, (n_slabs_per_w - 1) // 3, triple, 0)
    do_slab(n_slabs_per_w - 1, bufs[0], bufs[1], None, None)

    pltpu.sync_copy(pos_v.at[pl.ds(0, n_slabs_per_w * CHAINS_PER_SLAB)],
                    out_hbm.at[pl.ds(wid * (n_slabs_per_w * CHAINS_PER_SLAB),
                                     n_slabs_per_w * CHAINS_PER_SLAB)])


def kernel(initial_predictions, signals):
    if signals.ndim == 3:
        signals = jnp.squeeze(signals, axis=1)
    batch, num_peaks = initial_predictions.shape
    n_chains = batch * num_peaks
    n_slabs_per_w = batch // (NW * SLAB_ROWS)

    pos_flat = initial_predictions.reshape(-1)

    mesh = plsc.VectorSubcoreMesh(core_axis_name="c", subcore_axis_name="s")
    run = pl.kernel(
        functools.partial(_refine_body, n_slabs_per_w),
        out_type=jax.ShapeDtypeStruct((n_chains,), jnp.float32),
        mesh=mesh,
        compiler_params=pltpu.CompilerParams(use_tc_tiling_on_sc=True,
                                             needs_layout_passes=False),
        scratch_types=[
            pltpu.VMEM((n_slabs_per_w * CHAINS_PER_SLAB + L,), jnp.float32),
            pltpu.VMEM((HALF_ROWS, SIGNAL_LENGTH), jnp.float32),
            pltpu.VMEM((HALF_ROWS, SIGNAL_LENGTH), jnp.float32),
            pltpu.VMEM((HALF_ROWS, SIGNAL_LENGTH), jnp.float32),
            pltpu.SemaphoreType.DMA,
        ],
    )
    out = run(pos_flat, signals)
    return out.reshape(batch, num_peaks)
